# Initial kernel scaffold; baseline (speedup 1.0000x reference)
import jax
import jax.numpy as jnp
from jax.experimental import pallas as pl


def kernel(U, I, spatial_ker_weights, bilateral_ker_weights, compatibility_matrix):
    raise NotImplementedError("write your pallas kernel here")


# speedup vs baseline: 2.4493x; 2.4493x over previous
import functools

import jax
import jax.numpy as jnp
from jax.experimental import pallas as pl
from jax.experimental.pallas import tpu as pltpu

L = 16
D = H = W = 16
N = D * H * W
ALPHA = 80.0
BETA = 0.5
GAMMA = 3.0

TJ = 512
TI = 512
NJ = N // TJ
NI = N // TI

_A = -1.0 / (2.0 * ALPHA * ALPHA)
_B = -1.0 / (2.0 * BETA * BETA)
_C = -1.0 / (2.0 * GAMMA * GAMMA)


def _voxel_zyx(idx):
    z = (idx >> 8).astype(jnp.float32)
    y = ((idx >> 4) & 15).astype(jnp.float32)
    x = (idx & 15).astype(jnp.float32)
    return z, y, x


def _crf_kernel(u_ref, f_ref, sw_ref, bw_ref, cm_ref, out_ref):
    j0 = pl.program_id(0) * TJ

    jidx = jax.lax.broadcasted_iota(jnp.int32, (1, TJ), 1) + j0
    zj, yj, xj = _voxel_zyx(jidx)
    fj = f_ref[0:1, pl.ds(j0, TJ)]
    sj = zj * zj + yj * yj + xj * xj
    hb = jnp.concatenate([
        (-2.0 * _A) * zj, (-2.0 * _A) * yj, (-2.0 * _A) * xj,
        (-2.0 * _B) * fj,
        jnp.ones_like(zj), jnp.zeros_like(zj),
        _A * sj + _B * fj * fj, jnp.zeros_like(zj),
    ], axis=0)
    hs = jnp.concatenate([
        (-2.0 * _C) * zj, (-2.0 * _C) * yj, (-2.0 * _C) * xj,
        jnp.zeros_like(zj),
        jnp.zeros_like(zj), jnp.ones_like(zj),
        _C * sj, jnp.zeros_like(zj),
    ], axis=0)
    hj = jnp.concatenate([hb, hs], axis=1)

    def body(i, acc):
        i0 = i * TI
        iidx = jax.lax.broadcasted_iota(jnp.int32, (TI, 1), 0) + i0
        zi, yi, xi = _voxel_zyx(iidx)
        fi = f_ref[0:1, pl.ds(i0, TI)].reshape(TI, 1)
        si = zi * zi + yi * yi + xi * xi
        gi = jnp.concatenate([
            zi, yi, xi, fi,
            _A * si + _B * fi * fi,
            _C * si,
            jnp.ones_like(zi), jnp.zeros_like(zi),
        ], axis=1)

        arg = jax.lax.dot_general(
            gi, hj, (((1,), (0,)), ((), ())),
            preferred_element_type=jnp.float32)
        k_tile = jnp.exp(arg)

        u_i = u_ref[:, pl.ds(i0, TI)]
        m = jnp.max(u_i, axis=0, keepdims=True)
        e = jnp.exp(u_i - m)
        qs = e / jnp.sum(e, axis=0, keepdims=True)
        a_i = jnp.concatenate(
            [qs, jnp.ones((1, TI), jnp.float32)], axis=0)

        return acc + jax.lax.dot_general(
            a_i, k_tile, (((1,), (0,)), ((), ())),
            preferred_element_type=jnp.float32)

    acc = jax.lax.fori_loop(
        0, NI, body, jnp.zeros((L + 1, 2 * TJ), jnp.float32))

    yb = acc[:L, :TJ] / acc[L:L + 1, :TJ]
    ys = acc[:L, TJ:] / acc[L:L + 1, TJ:]
    m_msg = (jnp.dot(sw_ref[...], ys, preferred_element_type=jnp.float32)
             + jnp.dot(bw_ref[...], yb, preferred_element_type=jnp.float32))
    m_msg = jnp.dot(cm_ref[...], m_msg, preferred_element_type=jnp.float32)
    q = u_ref[:, pl.ds(j0, TJ)] + m_msg
    mx = jnp.max(q, axis=0, keepdims=True)
    eq = jnp.exp(q - mx)
    out_ref[...] = eq / jnp.sum(eq, axis=0, keepdims=True)


@jax.jit
def kernel(U, I, spatial_ker_weights, bilateral_ker_weights,
           compatibility_matrix):
    u_flat = U[0].reshape(L, N)
    feat = I.reshape(1, N)
    out = pl.pallas_call(
        _crf_kernel,
        grid=(NJ,),
        in_specs=[
            pl.BlockSpec((L, N), lambda j: (0, 0)),
            pl.BlockSpec((1, N), lambda j: (0, 0)),
            pl.BlockSpec((L, L), lambda j: (0, 0)),
            pl.BlockSpec((L, L), lambda j: (0, 0)),
            pl.BlockSpec((L, L), lambda j: (0, 0)),
        ],
        out_specs=pl.BlockSpec((L, TJ), lambda j: (0, j)),
        out_shape=jax.ShapeDtypeStruct((L, N), jnp.float32),
    )(u_flat, feat, spatial_ker_weights, bilateral_ker_weights,
      compatibility_matrix)
    return out.reshape(1, L, D, H, W)
